# Initial kernel scaffold; baseline (speedup 1.0000x reference)
import jax
import jax.numpy as jnp
from jax.experimental import pallas as pl


def kernel(params, edge_index, gate, forward_level, forward_index, backward_level):
    raise NotImplementedError("write your pallas kernel here")


# speedup vs baseline: 12.3178x; 12.3178x over previous
import functools

import jax
import jax.numpy as jnp
from jax import lax
from jax.experimental import pallas as pl
from jax.experimental.pallas import tpu as pltpu
from jax.experimental.pallas import tpu_sc as plsc

N = 10000
E = 160000
H = 128
DM = 32
D = 144
NC = 2
NS = 16
NW = NC * NS
CH = 128
N_PAD = 10240
E_PAD = 163840
EPT = E_PAD // NW
NCH = EPT // CH
RPT = N_PAD // NS
_PREC = lax.Precision.DEFAULT


def _ek_body(x1_ref, x2_ref, wk1, wk2, bk, wa2, out_ref):
    k = (jnp.dot(x1_ref[...], wk1[...], precision=_PREC)
         + jnp.dot(x2_ref[...], wk2[...], precision=_PREC) + bk[...])
    ak = jnp.dot(k, wa2[...], precision=_PREC)
    out_ref[...] = jnp.exp(ak - jnp.max(ak))


_ek_call = pl.pallas_call(
    _ek_body,
    out_shape=jax.ShapeDtypeStruct((N, 1), jnp.float32),
)

_TBR = 640
_TG = N_PAD // _TBR


def _table_body(x1_ref, x2_ref, ek_ref, wv1, wv2, bv, out_ref):
    i = pl.program_id(0)
    v = (jnp.dot(x1_ref[...], wv1[...], precision=_PREC)
         + jnp.dot(x2_ref[...], wv2[...], precision=_PREC) + bv[...])
    ek = ek_ref[...]
    val = jnp.concatenate(
        [ek * v, ek, jnp.zeros((_TBR, D - H - 1), jnp.float32)], axis=1)
    rows = i * _TBR + lax.broadcasted_iota(jnp.int32, (_TBR, 1), 0)
    out_ref[...] = jnp.where(rows < N, val, 0.0)


def _attn_table(x1, x2, wk1, wk2, bk, wa2, wv1, wv2, bv):
    ek = _ek_call(x1, x2, wk1, wk2, bk, wa2)
    return pl.pallas_call(
        _table_body,
        grid=(_TG,),
        in_specs=[
            pl.BlockSpec((_TBR, x1.shape[1]), lambda i: (i, 0)),
            pl.BlockSpec((_TBR, x2.shape[1]), lambda i: (i, 0)),
            pl.BlockSpec((_TBR, 1), lambda i: (i, 0)),
            pl.BlockSpec(wv1.shape, lambda i: (0, 0)),
            pl.BlockSpec(wv2.shape, lambda i: (0, 0)),
            pl.BlockSpec((1, H), lambda i: (0, 0)),
        ],
        out_specs=pl.BlockSpec((_TBR, D), lambda i: (i, 0)),
        out_shape=jax.ShapeDtypeStruct((N_PAD, D), jnp.float32),
    )(x1, x2, ek, wv1, wv2, bv)


def _segsum_kernel(table_hbm, src_hbm, dst_hbm, out_hbm,
                   src_v, dst_v, rows_v, acc, sem):
    cid = lax.axis_index("c")
    sid = lax.axis_index("s")
    wid = sid * NC + cid
    base = sid * RPT

    z16 = jnp.zeros((16,), jnp.float32)

    def _zrow(r, _):
        def _zcol(j, _):
            rows_v[r, pl.ds(j * 16, 16)] = z16
            return 0
        return lax.fori_loop(0, D // 16, _zcol, 0)

    lax.fori_loop(0, CH, _zrow, 0)

    def _zcopy(j, _):
        pltpu.sync_copy(rows_v, acc.at[pl.ds(base + j * CH, CH)])
        return 0

    lax.fori_loop(0, RPT // CH, _zcopy, 0)

    pltpu.sync_copy(src_hbm.at[wid], src_v)
    pltpu.sync_copy(dst_hbm.at[wid], dst_v)
    plsc.subcore_barrier()

    def _chunk(c, _):
        pltpu.async_copy(table_hbm.at[src_v.at[c]], rows_v, sem).wait()
        pltpu.sync_copy(rows_v, acc.at[dst_v.at[c]], add=True)
        return 0

    lax.fori_loop(0, NCH, _chunk, 0)
    plsc.subcore_barrier()

    pltpu.sync_copy(acc.at[pl.ds(base, RPT)],
                    out_hbm.at[cid, pl.ds(base, RPT)])


@functools.cache
def _segsum_call():
    return functools.partial(
        pl.kernel,
        out_type=jax.ShapeDtypeStruct((NC, N_PAD, D), jnp.float32),
        mesh=plsc.VectorSubcoreMesh(core_axis_name="c", subcore_axis_name="s",
                                    num_cores=NC, num_subcores=NS),
        compiler_params=pltpu.CompilerParams(use_tc_tiling_on_sc=False),
        scratch_types=[
            pltpu.VMEM((NCH, CH), jnp.int32),
            pltpu.VMEM((NCH, CH), jnp.int32),
            pltpu.VMEM((CH, D), jnp.float32),
            pltpu.VMEM_SHARED((N_PAD, D), jnp.float32),
            pltpu.SemaphoreType.DMA,
        ],
    )(_segsum_kernel)


def _segsum(table, srcp, dstp):
    return _segsum_call()(table, srcp, dstp)


def _gru_body(level, gval, t_ref, h_ref, g_ref, fl_ref,
              wi, bi, wh, bh, out_ref):
    t = t_ref[0] + t_ref[1]
    s = t[:, H:H + 1]
    pos = s > 0
    msg = jnp.where(pos, t[:, :H] / jnp.where(pos, s, 1.0), 0.0)
    h = h_ref[...]
    gi = jnp.dot(msg, wi[...], precision=_PREC) + bi[...]
    gh = jnp.dot(h, wh[...], precision=_PREC) + bh[...]
    r = jax.nn.sigmoid(gi[:, :H] + gh[:, :H])
    z = jax.nn.sigmoid(gi[:, H:2 * H] + gh[:, H:2 * H])
    ng = jnp.tanh(gi[:, 2 * H:] + r * gh[:, 2 * H:])
    hn = (1.0 - z) * ng + z * h
    m = (fl_ref[...] == level) & (g_ref[...] == gval)
    out_ref[...] = jnp.where(m, hn, h)


_GRU_G = 5
_BR = N // _GRU_G


def _gru_call(level, gval, t, h, gate2, fl2, gp):
    body = functools.partial(_gru_body, level, gval)
    return pl.pallas_call(
        body,
        grid=(_GRU_G,),
        in_specs=[
            pl.BlockSpec((NC, _BR, D), lambda i: (0, i, 0)),
            pl.BlockSpec((_BR, H), lambda i: (i, 0)),
            pl.BlockSpec((_BR, 1), lambda i: (i, 0)),
            pl.BlockSpec((_BR, 1), lambda i: (i, 0)),
            pl.BlockSpec((H, 3 * H), lambda i: (0, 0)),
            pl.BlockSpec((1, 3 * H), lambda i: (0, 0)),
            pl.BlockSpec((H, 3 * H), lambda i: (0, 0)),
            pl.BlockSpec((1, 3 * H), lambda i: (0, 0)),
        ],
        out_specs=pl.BlockSpec((_BR, H), lambda i: (i, 0)),
        out_shape=jax.ShapeDtypeStruct((N, H), jnp.float32),
    )(t, h, gate2, fl2, gp['Wi'], gp['bi'].reshape(1, 3 * H),
      gp['Wh'], gp['bh'].reshape(1, 3 * H))


def _mlp_body(x_ref, w0, b0, g0, be0, w1, b1, g1, be1, w2, b2, out_ref):
    def bn(y, g, b):
        m = jnp.mean(y, axis=0, keepdims=True)
        v = jnp.mean((y - m) ** 2, axis=0, keepdims=True)
        return (y - m) / jnp.sqrt(v + 1e-5) * g + b

    y = jnp.dot(x_ref[...], w0[...], precision=_PREC) + b0[...]
    y = jax.nn.relu(bn(y, g0[...], be0[...]))
    y = jnp.dot(y, w1[...], precision=_PREC) + b1[...]
    y = jax.nn.relu(bn(y, g1[...], be1[...]))
    out_ref[...] = jnp.dot(y, w2[...], precision=_PREC) + b2[...]


_mlp_call = pl.pallas_call(
    _mlp_body,
    out_shape=jax.ShapeDtypeStruct((N, 1), jnp.float32),
)


def kernel(params, edge_index, gate, forward_level, forward_index,
           backward_level):
    p = params
    f32 = jnp.float32
    src = edge_index[0].astype(jnp.int32)
    dst = edge_index[1].astype(jnp.int32)
    padlen = E_PAD - E
    pad_idx = jnp.full((padlen,), N, jnp.int32)
    srcp = jnp.concatenate([src, pad_idx]).reshape(NW, NCH, CH)
    dstp = jnp.concatenate([dst, pad_idx]).reshape(NW, NCH, CH)
    gate2 = gate.astype(jnp.int32).reshape(N, 1)
    fl2 = forward_level.astype(jnp.int32).reshape(N, 1)

    hs = jnp.zeros((N, H), f32)
    hf = jnp.tile(jnp.ones((1, 1), f32) @ p['hf_W'] + p['hf_b'], (N, 1))

    zH = jnp.zeros((H, H), f32)

    def attn(x1, x2, tf, split):
        if split:
            wk1, wk2 = tf['Wk'][:H], tf['Wk'][H:]
            wv1, wv2 = tf['Wv'][:H], tf['Wv'][H:]
        else:
            wk1, wk2 = tf['Wk'], zH
            wv1, wv2 = tf['Wv'], zH
        table = _attn_table(x1, x2, wk1, wk2, tf['bk'].reshape(1, H),
                            tf['Wa'][H:], wv1, wv2, tf['bv'].reshape(1, H))
        return _segsum(table, srcp, dstp)

    for level in (1, 2):
        t_a = attn(hs, hs, p['and_strc'], False)
        t_b = attn(hs, hf, p['and_func'], True)
        hs1 = _gru_call(level, 6, t_a, hs, gate2, fl2, p['g_and_strc'])
        hf1 = _gru_call(level, 6, t_b, hf, gate2, fl2, p['g_and_func'])
        t_c = attn(hs1, hs1, p['not_strc'], False)
        hs = _gru_call(level, 4, t_c, hs1, gate2, fl2, p['g_not_strc'])
        t_d = attn(hf1, hf1, p['not_func'], False)
        hf = _gru_call(level, 4, t_d, hf1, gate2, fl2, p['g_not_func'])

    mp = p['mlp']
    return _mlp_call(hf, mp['W0'], mp['b0'].reshape(1, DM),
                     mp['g0'].reshape(1, DM), mp['be0'].reshape(1, DM),
                     mp['W1'], mp['b1'].reshape(1, DM),
                     mp['g1'].reshape(1, DM), mp['be1'].reshape(1, DM),
                     mp['W2'], mp['b2'].reshape(1, 1))


# speedup vs baseline: 13.5143x; 1.0971x over previous
import functools

import jax
import jax.numpy as jnp
from jax import lax
from jax.experimental import pallas as pl
from jax.experimental.pallas import tpu as pltpu
from jax.experimental.pallas import tpu_sc as plsc

N = 10000
E = 160000
H = 128
DM = 32
D = 144
NC = 2
NS = 16
NW = NC * NS
CH = 64
N_PAD = 10240
E_PAD = 163840
EPT = E_PAD // NW
NCH = EPT // CH
RPT = N_PAD // NS
_PREC = lax.Precision.DEFAULT


def _ek_body(x1_ref, x2_ref, wk1, wk2, bk, wa2, out_ref):
    k = (jnp.dot(x1_ref[...], wk1[...], precision=_PREC)
         + jnp.dot(x2_ref[...], wk2[...], precision=_PREC) + bk[...])
    ak = jnp.dot(k, wa2[...], precision=_PREC)
    out_ref[...] = jnp.exp(ak - jnp.max(ak))


_ek_call = pl.pallas_call(
    _ek_body,
    out_shape=jax.ShapeDtypeStruct((N, 1), jnp.float32),
)

_TBR = 640
_TG = N_PAD // _TBR


def _table_body(x1_ref, x2_ref, ek_ref, wv1, wv2, bv, out_ref):
    i = pl.program_id(0)
    v = (jnp.dot(x1_ref[...], wv1[...], precision=_PREC)
         + jnp.dot(x2_ref[...], wv2[...], precision=_PREC) + bv[...])
    ek = ek_ref[...]
    val = jnp.concatenate(
        [ek * v, ek, jnp.zeros((_TBR, D - H - 1), jnp.float32)], axis=1)
    rows = i * _TBR + lax.broadcasted_iota(jnp.int32, (_TBR, 1), 0)
    out_ref[...] = jnp.where(rows < N, val, 0.0)


def _attn_table(x1, x2, wk1, wk2, bk, wa2, wv1, wv2, bv):
    ek = _ek_call(x1, x2, wk1, wk2, bk, wa2)
    return pl.pallas_call(
        _table_body,
        grid=(_TG,),
        in_specs=[
            pl.BlockSpec((_TBR, x1.shape[1]), lambda i: (i, 0)),
            pl.BlockSpec((_TBR, x2.shape[1]), lambda i: (i, 0)),
            pl.BlockSpec((_TBR, 1), lambda i: (i, 0)),
            pl.BlockSpec(wv1.shape, lambda i: (0, 0)),
            pl.BlockSpec(wv2.shape, lambda i: (0, 0)),
            pl.BlockSpec((1, H), lambda i: (0, 0)),
        ],
        out_specs=pl.BlockSpec((_TBR, D), lambda i: (i, 0)),
        out_shape=jax.ShapeDtypeStruct((N_PAD, D), jnp.float32),
    )(x1, x2, ek, wv1, wv2, bv)


def _segsum_kernel(table_hbm, src_hbm, dst_hbm, out_hbm,
                   src_v, dst_v, rows_v, rows_w, acc, sem, sem2):
    cid = lax.axis_index("c")
    sid = lax.axis_index("s")
    wid = sid * NC + cid
    base = sid * RPT

    z16 = jnp.zeros((16,), jnp.float32)

    def _zrow(r, _):
        def _zcol(j, _):
            rows_v[r, pl.ds(j * 16, 16)] = z16
            return 0
        return lax.fori_loop(0, D // 16, _zcol, 0)

    lax.fori_loop(0, CH, _zrow, 0)

    def _zcopy(j, _):
        pltpu.sync_copy(rows_v, acc.at[pl.ds(base + j * CH, CH)])
        return 0

    lax.fori_loop(0, RPT // CH, _zcopy, 0)

    pltpu.sync_copy(src_hbm.at[wid], src_v)
    pltpu.sync_copy(dst_hbm.at[wid], dst_v)
    plsc.subcore_barrier()

    pltpu.async_copy(table_hbm.at[src_v.at[0]], rows_v, sem)

    def _pair(i, _):
        c0 = 2 * i
        pltpu.async_copy(table_hbm.at[src_v.at[c0 + 1]], rows_w, sem2)
        pltpu.make_async_copy(table_hbm.at[src_v.at[c0]], rows_v, sem).wait()
        pltpu.sync_copy(rows_v, acc.at[dst_v.at[c0]], add=True)

        @pl.when(i < NCH // 2 - 1)
        def _():
            pltpu.async_copy(table_hbm.at[src_v.at[c0 + 2]], rows_v, sem)

        pltpu.make_async_copy(table_hbm.at[src_v.at[c0 + 1]],
                              rows_w, sem2).wait()
        pltpu.sync_copy(rows_w, acc.at[dst_v.at[c0 + 1]], add=True)
        return 0

    lax.fori_loop(0, NCH // 2, _pair, 0)
    plsc.subcore_barrier()

    pltpu.sync_copy(acc.at[pl.ds(base, RPT)],
                    out_hbm.at[cid, pl.ds(base, RPT)])


@functools.cache
def _segsum_call():
    return functools.partial(
        pl.kernel,
        out_type=jax.ShapeDtypeStruct((NC, N_PAD, D), jnp.float32),
        mesh=plsc.VectorSubcoreMesh(core_axis_name="c", subcore_axis_name="s",
                                    num_cores=NC, num_subcores=NS),
        compiler_params=pltpu.CompilerParams(use_tc_tiling_on_sc=False),
        scratch_types=[
            pltpu.VMEM((NCH, CH), jnp.int32),
            pltpu.VMEM((NCH, CH), jnp.int32),
            pltpu.VMEM((CH, D), jnp.float32),
            pltpu.VMEM((CH, D), jnp.float32),
            pltpu.VMEM_SHARED((N_PAD, D), jnp.float32),
            pltpu.SemaphoreType.DMA,
            pltpu.SemaphoreType.DMA,
        ],
    )(_segsum_kernel)


def _segsum(table, srcp, dstp):
    return _segsum_call()(table, srcp, dstp)


def _gru_body(level, gval, t_ref, h_ref, g_ref, fl_ref,
              wi, bi, wh, bh, out_ref):
    t = t_ref[0] + t_ref[1]
    s = t[:, H:H + 1]
    pos = s > 0
    msg = jnp.where(pos, t[:, :H] / jnp.where(pos, s, 1.0), 0.0)
    h = h_ref[...]
    gi = jnp.dot(msg, wi[...], precision=_PREC) + bi[...]
    gh = jnp.dot(h, wh[...], precision=_PREC) + bh[...]
    r = jax.nn.sigmoid(gi[:, :H] + gh[:, :H])
    z = jax.nn.sigmoid(gi[:, H:2 * H] + gh[:, H:2 * H])
    ng = jnp.tanh(gi[:, 2 * H:] + r * gh[:, 2 * H:])
    hn = (1.0 - z) * ng + z * h
    m = (fl_ref[...] == level) & (g_ref[...] == gval)
    out_ref[...] = jnp.where(m, hn, h)


_GRU_G = 5
_BR = N // _GRU_G


def _gru_call(level, gval, t, h, gate2, fl2, gp):
    body = functools.partial(_gru_body, level, gval)
    return pl.pallas_call(
        body,
        grid=(_GRU_G,),
        in_specs=[
            pl.BlockSpec((NC, _BR, D), lambda i: (0, i, 0)),
            pl.BlockSpec((_BR, H), lambda i: (i, 0)),
            pl.BlockSpec((_BR, 1), lambda i: (i, 0)),
            pl.BlockSpec((_BR, 1), lambda i: (i, 0)),
            pl.BlockSpec((H, 3 * H), lambda i: (0, 0)),
            pl.BlockSpec((1, 3 * H), lambda i: (0, 0)),
            pl.BlockSpec((H, 3 * H), lambda i: (0, 0)),
            pl.BlockSpec((1, 3 * H), lambda i: (0, 0)),
        ],
        out_specs=pl.BlockSpec((_BR, H), lambda i: (i, 0)),
        out_shape=jax.ShapeDtypeStruct((N, H), jnp.float32),
    )(t, h, gate2, fl2, gp['Wi'], gp['bi'].reshape(1, 3 * H),
      gp['Wh'], gp['bh'].reshape(1, 3 * H))


def _mlp_body(x_ref, w0, b0, g0, be0, w1, b1, g1, be1, w2, b2, out_ref):
    def bn(y, g, b):
        m = jnp.mean(y, axis=0, keepdims=True)
        v = jnp.mean((y - m) ** 2, axis=0, keepdims=True)
        return (y - m) / jnp.sqrt(v + 1e-5) * g + b

    y = jnp.dot(x_ref[...], w0[...], precision=_PREC) + b0[...]
    y = jax.nn.relu(bn(y, g0[...], be0[...]))
    y = jnp.dot(y, w1[...], precision=_PREC) + b1[...]
    y = jax.nn.relu(bn(y, g1[...], be1[...]))
    out_ref[...] = jnp.dot(y, w2[...], precision=_PREC) + b2[...]


_mlp_call = pl.pallas_call(
    _mlp_body,
    out_shape=jax.ShapeDtypeStruct((N, 1), jnp.float32),
)


def kernel(params, edge_index, gate, forward_level, forward_index,
           backward_level):
    p = params
    f32 = jnp.float32
    src = edge_index[0].astype(jnp.int32)
    dst = edge_index[1].astype(jnp.int32)
    padlen = E_PAD - E
    pad_idx = jnp.full((padlen,), N, jnp.int32)
    srcp = jnp.concatenate([src, pad_idx]).reshape(NW, NCH, CH)
    dstp = jnp.concatenate([dst, pad_idx]).reshape(NW, NCH, CH)
    gate2 = gate.astype(jnp.int32).reshape(N, 1)
    fl2 = forward_level.astype(jnp.int32).reshape(N, 1)

    hs = jnp.zeros((N, H), f32)
    hf = jnp.tile(jnp.ones((1, 1), f32) @ p['hf_W'] + p['hf_b'], (N, 1))

    zH = jnp.zeros((H, H), f32)

    def attn(x1, x2, tf, split):
        if split:
            wk1, wk2 = tf['Wk'][:H], tf['Wk'][H:]
            wv1, wv2 = tf['Wv'][:H], tf['Wv'][H:]
        else:
            wk1, wk2 = tf['Wk'], zH
            wv1, wv2 = tf['Wv'], zH
        table = _attn_table(x1, x2, wk1, wk2, tf['bk'].reshape(1, H),
                            tf['Wa'][H:], wv1, wv2, tf['bv'].reshape(1, H))
        return _segsum(table, srcp, dstp)

    for level in (1, 2):
        t_a = attn(hs, hs, p['and_strc'], False)
        t_b = attn(hs, hf, p['and_func'], True)
        hs1 = _gru_call(level, 6, t_a, hs, gate2, fl2, p['g_and_strc'])
        hf1 = _gru_call(level, 6, t_b, hf, gate2, fl2, p['g_and_func'])
        t_c = attn(hs1, hs1, p['not_strc'], False)
        hs = _gru_call(level, 4, t_c, hs1, gate2, fl2, p['g_not_strc'])
        t_d = attn(hf1, hf1, p['not_func'], False)
        hf = _gru_call(level, 4, t_d, hf1, gate2, fl2, p['g_not_func'])

    mp = p['mlp']
    return _mlp_call(hf, mp['W0'], mp['b0'].reshape(1, DM),
                     mp['g0'].reshape(1, DM), mp['be0'].reshape(1, DM),
                     mp['W1'], mp['b1'].reshape(1, DM),
                     mp['g1'].reshape(1, DM), mp['be1'].reshape(1, DM),
                     mp['W2'], mp['b2'].reshape(1, 1))
